# Initial kernel scaffold; baseline (speedup 1.0000x reference)
import jax
import jax.numpy as jnp
from jax.experimental import pallas as pl


def kernel(Eu, Ev, edge_index):
    raise NotImplementedError("write your pallas kernel here")


# speedup vs baseline: 1.3396x; 1.3396x over previous
import functools

import jax
import jax.numpy as jnp
from jax import lax
from jax.experimental import pallas as pl
from jax.experimental.pallas import tpu as pltpu
from jax.experimental.pallas import tpu_sc as plsc

N_NODES = 10000
N_EDGES = 320000
D_FEAT = 128

NUM_CORES = 2
NUM_SUBCORES = 16
NW = NUM_CORES * NUM_SUBCORES
EPW = N_EDGES // NW
CHUNK = 80
NCHUNKS = EPW // CHUNK
LANES = 16
GRPS = CHUNK // LANES
UNROLL = 8


def _edge_scores_kernel(eu_hbm, ev_hbm, src_hbm, dst_hbm, out_hbm,
                        sidx, didx, ua, va, ub, vb, outv, sem_a, sem_b):
    wid = lax.axis_index("s") * NUM_CORES + lax.axis_index("c")
    base = wid * EPW

    pltpu.sync_copy(src_hbm.at[pl.ds(base, EPW)], sidx)
    pltpu.sync_copy(dst_hbm.at[pl.ds(base, EPW)], didx)

    def issue(g, ur, vr, sem):
        off = g * CHUNK
        pltpu.async_copy(eu_hbm.at[sidx.at[pl.ds(off, CHUNK)]], ur, sem)
        pltpu.async_copy(ev_hbm.at[didx.at[pl.ds(off, CHUNK)]], vr, sem)

    def wait(ur, vr, sem):
        pltpu.make_async_copy(eu_hbm.at[sidx.at[pl.ds(0, CHUNK)]],
                              ur, sem).wait()
        pltpu.make_async_copy(ev_hbm.at[didx.at[pl.ds(0, CHUNK)]],
                              vr, sem).wait()

    def compute(g, ur, vr):
        for grp in range(GRPS):
            row0 = jnp.arange(LANES, dtype=jnp.int32) + grp * LANES

            def f_block(_, c):
                a0, a1, col = c
                for j in range(UNROLL):
                    cj = col + j
                    uu = plsc.load_gather(ur, [row0, cj])
                    vv = plsc.load_gather(vr, [row0, cj])
                    if j % 2 == 0:
                        a0 = a0 + uu * vv
                    else:
                        a1 = a1 + uu * vv
                return a0, a1, col + UNROLL

            zf = jnp.zeros((LANES,), jnp.float32)
            zi = jnp.zeros((LANES,), jnp.int32)
            a0, a1, _ = lax.fori_loop(0, D_FEAT // UNROLL, f_block,
                                      (zf, zf, zi))
            acc = a0 + a1
            sig = 1.0 / (1.0 + jnp.exp(-acc))
            outv[pl.ds(g * CHUNK + grp * LANES, LANES)] = sig

    issue(0, ua, va, sem_a)

    def body(i, carry):
        g0 = 2 * i
        issue(g0 + 1, ub, vb, sem_b)
        wait(ua, va, sem_a)
        compute(g0, ua, va)
        issue(g0 + 2, ua, va, sem_a)
        wait(ub, vb, sem_b)
        compute(g0 + 1, ub, vb)
        return carry

    lax.fori_loop(0, (NCHUNKS - 1) // 2, body, 0)
    wait(ua, va, sem_a)
    compute(NCHUNKS - 1, ua, va)

    pltpu.sync_copy(outv, out_hbm.at[pl.ds(base, EPW)])


@jax.jit
def _edge_scores(eu, ev, src, dst):
    mesh = plsc.VectorSubcoreMesh(core_axis_name="c", subcore_axis_name="s")
    run = functools.partial(
        pl.kernel,
        mesh=mesh,
        out_type=jax.ShapeDtypeStruct((N_EDGES,), jnp.float32),
        scratch_types=[
            pltpu.VMEM((EPW,), jnp.int32),
            pltpu.VMEM((EPW,), jnp.int32),
            pltpu.VMEM((CHUNK, D_FEAT), jnp.float32),
            pltpu.VMEM((CHUNK, D_FEAT), jnp.float32),
            pltpu.VMEM((CHUNK, D_FEAT), jnp.float32),
            pltpu.VMEM((CHUNK, D_FEAT), jnp.float32),
            pltpu.VMEM((EPW,), jnp.float32),
            pltpu.SemaphoreType.DMA,
            pltpu.SemaphoreType.DMA,
        ],
        compiler_params=pltpu.CompilerParams(needs_layout_passes=False),
    )(_edge_scores_kernel)
    return run(eu, ev, src, dst)


def kernel(Eu, Ev, edge_index):
    src = edge_index[0].astype(jnp.int32)
    dst = edge_index[1].astype(jnp.int32)
    return _edge_scores(Eu, Ev, src, dst)
